# baseline (speedup 1.0000x reference)
import functools

import jax
import jax.numpy as jnp
from jax import lax
from jax.experimental import pallas as pl
from jax.experimental.pallas import tpu as pltpu
from jax.experimental.pallas import tpu_sc as plsc

B = 64
LAYERS = 3
STEPS = 3

NC = 2
NS = 16
NW = NC * NS


def _sc_mesh():
    return plsc.VectorSubcoreMesh(core_axis_name="c", subcore_axis_name="s",
                                  num_cores=NC, num_subcores=NS)


@functools.partial(jax.jit, static_argnames=("e_pad", "d"))
def _sc_gather(h, src2d, *, e_pad, d):
    epw = e_pad // NW
    ch = epw // 128

    @functools.partial(
        pl.kernel,
        mesh=_sc_mesh(),
        out_type=jax.ShapeDtypeStruct((e_pad, d), jnp.float32),
        scratch_types=[
            pltpu.VMEM((ch, 128), jnp.int32),
            pltpu.VMEM((128, d), jnp.float32),
            pltpu.SemaphoreType.DMA,
        ],
        compiler_params=pltpu.CompilerParams(use_tc_tiling_on_sc=False),
        name="mpnn_sc_gather",
    )
    def k(h_hbm, src_hbm, out_hbm, idx_v, rows_v, sem):
        wid = lax.axis_index("c") * NS + lax.axis_index("s")
        pltpu.sync_copy(src_hbm.at[pl.ds(wid * ch, ch)], idx_v)

        def body(j, carry):
            pltpu.async_copy(h_hbm.at[idx_v.at[j]], rows_v, sem).wait()
            pltpu.sync_copy(rows_v, out_hbm.at[pl.ds(wid * epw + j * 128, 128)])
            return carry

        lax.fori_loop(0, ch, body, 0)

    return k(h, src2d)


@functools.partial(jax.jit, static_argnames=("e_pad", "n_pad", "d"))
def _sc_scatter_add(msg, dst2d, zeros_nd, *, e_pad, n_pad, d):
    epw = e_pad // NW
    ch = epw // 128
    npt = n_pad // NS

    @functools.partial(
        pl.kernel,
        mesh=_sc_mesh(),
        out_type=jax.ShapeDtypeStruct((NC, n_pad, d), jnp.float32),
        scratch_types=[
            pltpu.VMEM((ch, 128), jnp.int32),
            pltpu.VMEM((128, d), jnp.float32),
            pltpu.VMEM_SHARED((n_pad, d), jnp.float32),
            pltpu.SemaphoreType.DMA,
        ],
        compiler_params=pltpu.CompilerParams(use_tc_tiling_on_sc=False),
        name="mpnn_sc_scatter",
    )
    def k(msg_hbm, dst_hbm, zero_hbm, out_hbm, idx_v, msg_v, acc_sh, sem):
        cid = lax.axis_index("c")
        sid = lax.axis_index("s")
        wid = cid * NS + sid
        pltpu.sync_copy(zero_hbm.at[pl.ds(sid * npt, npt)],
                        acc_sh.at[pl.ds(sid * npt, npt)])
        plsc.subcore_barrier()
        pltpu.sync_copy(dst_hbm.at[pl.ds(wid * ch, ch)], idx_v)

        def body(j, carry):
            pltpu.async_copy(msg_hbm.at[pl.ds(wid * epw + j * 128, 128)],
                             msg_v, sem).wait()
            pltpu.sync_copy(msg_v, acc_sh.at[idx_v.at[j]], add=True)
            return carry

        lax.fori_loop(0, ch, body, 0)
        plsc.subcore_barrier()
        pltpu.sync_copy(acc_sh.at[pl.ds(sid * npt, npt)],
                        out_hbm.at[cid, pl.ds(sid * npt, npt)])

    return k(msg, dst2d, zeros_nd)


def _msg_body(attr_ref, hs_ref, ew1_ref, eb1_ref, w2r_ref, eb2_ref, out_ref):
    eh = jax.nn.relu(
        jnp.dot(attr_ref[...], ew1_ref[...],
                preferred_element_type=jnp.float32) + eb1_ref[...])
    hs = hs_ref[...]
    et = eh.shape[0]
    u = (eh[:, :, None] * hs[:, None, :]).reshape(et, -1)
    out_ref[...] = (
        jnp.dot(u, w2r_ref[...], preferred_element_type=jnp.float32)
        + jnp.dot(hs, eb2_ref[...], preferred_element_type=jnp.float32))


def _tc_messages(attr_p, hs, ew1, eb1_2d, w2r, eb2_2d, *, e_pad, d, f_edge, h_edge):
    et = 1024
    grid = e_pad // et
    return pl.pallas_call(
        _msg_body,
        grid=(grid,),
        in_specs=[
            pl.BlockSpec((et, f_edge), lambda i: (i, 0)),
            pl.BlockSpec((et, d), lambda i: (i, 0)),
            pl.BlockSpec((f_edge, h_edge), lambda i: (0, 0)),
            pl.BlockSpec((1, h_edge), lambda i: (0, 0)),
            pl.BlockSpec((h_edge * d, d), lambda i: (0, 0)),
            pl.BlockSpec((d, d), lambda i: (0, 0)),
        ],
        out_specs=pl.BlockSpec((et, d), lambda i: (i, 0)),
        out_shape=jax.ShapeDtypeStruct((e_pad, d), jnp.float32),
    )(attr_p, hs, ew1, eb1_2d, w2r, eb2_2d)


def _update_body(p0_ref, p1_ref, h_ref, rw_ref, cb_ref, wih_ref, whh_ref,
                 bih_ref, bhh_ref, out_ref):
    h = h_ref[...]
    d = h.shape[1]
    agg = (p0_ref[...] + p1_ref[...]
           + jnp.dot(h, rw_ref[...], preferred_element_type=jnp.float32)
           + cb_ref[...])
    gi = jnp.dot(agg, wih_ref[...], preferred_element_type=jnp.float32) + bih_ref[...]
    gh = jnp.dot(h, whh_ref[...], preferred_element_type=jnp.float32) + bhh_ref[...]
    ir, iz, inn = gi[:, :d], gi[:, d:2 * d], gi[:, 2 * d:]
    hr, hz, hn = gh[:, :d], gh[:, d:2 * d], gh[:, 2 * d:]
    r = jax.nn.sigmoid(ir + hr)
    z = jax.nn.sigmoid(iz + hz)
    ng = jnp.tanh(inn + r * hn)
    out_ref[...] = (1.0 - z) * ng + z * h


def _tc_update(p0, p1, h, root_w, conv_b_2d, w_ih, w_hh, b_ih_2d, b_hh_2d, *, n, d):
    full = lambda *s: pl.BlockSpec(s, lambda: tuple(0 for _ in s))
    return pl.pallas_call(
        _update_body,
        in_specs=[full(n, d), full(n, d), full(n, d), full(d, d), full(1, d),
                  full(d, 3 * d), full(d, 3 * d), full(1, 3 * d), full(1, 3 * d)],
        out_specs=full(n, d),
        out_shape=jax.ShapeDtypeStruct((n, d), jnp.float32),
    )(p0, p1, h, root_w, conv_b_2d, w_ih, w_hh, b_ih_2d, b_hh_2d)


def _set2set_body(nr_ref, batch_ref, wih_ref, whh_ref, bi_ref, bh_ref,
                  row1_ref, rb1_ref, row2_ref, rb2_ref, out_ref):
    nr = nr_ref[...]
    d = nr.shape[1]
    bvec = batch_ref[...]
    gid = lax.broadcasted_iota(jnp.int32, (1, B), 1)
    m = (bvec == gid).astype(jnp.float32)
    q_star = jnp.zeros((B, 2 * d), dtype=jnp.float32)
    hl = jnp.zeros((B, d), dtype=jnp.float32)
    cl = jnp.zeros((B, d), dtype=jnp.float32)
    for _ in range(STEPS):
        g = (jnp.dot(q_star, wih_ref[...], preferred_element_type=jnp.float32)
             + jnp.dot(hl, whh_ref[...], preferred_element_type=jnp.float32)
             + bi_ref[...] + bh_ref[...])
        ig, fg, gg, og = (g[:, :d], g[:, d:2 * d], g[:, 2 * d:3 * d], g[:, 3 * d:])
        cl = jax.nn.sigmoid(fg) * cl + jax.nn.sigmoid(ig) * jnp.tanh(gg)
        hl = jax.nn.sigmoid(og) * jnp.tanh(cl)
        q = hl
        qn = jnp.dot(m, q, preferred_element_type=jnp.float32)
        e = jnp.sum(nr * qn, axis=-1, keepdims=True)
        emax = jnp.max(jnp.where(m > 0.0, e, -1e30), axis=0, keepdims=True)
        emax = jnp.where(emax < -1e29, 0.0, emax)
        ee = jnp.exp(e - jnp.sum(m * emax, axis=1, keepdims=True))
        denom = jnp.sum(m * ee, axis=0, keepdims=True)
        a = ee / (jnp.sum(m * denom, axis=1, keepdims=True) + 1e-16)
        rvec = lax.dot_general(m * a, nr, (((0,), (0,)), ((), ())),
                               preferred_element_type=jnp.float32)
        q_star = jnp.concatenate([q, rvec], axis=-1)
    hid = jax.nn.relu(
        jnp.dot(q_star, row1_ref[...], preferred_element_type=jnp.float32)
        + rb1_ref[...])
    out_ref[...] = (jnp.dot(hid, row2_ref[...], preferred_element_type=jnp.float32)
                    + rb2_ref[...])


def _tc_set2set(nr, batch_2d, w_ih, w_hh, bi_2d, bh_2d, ro_w1, rb1_2d, ro_w2,
                rb2_2d, *, n, d):
    full = lambda *s: pl.BlockSpec(s, lambda: tuple(0 for _ in s))
    return pl.pallas_call(
        _set2set_body,
        in_specs=[full(n, d), full(n, 1), full(2 * d, 4 * d), full(d, 4 * d),
                  full(1, 4 * d), full(1, 4 * d), full(2 * d, d), full(1, d),
                  full(d, d), full(1, d)],
        out_specs=full(B, d),
        out_shape=jax.ShapeDtypeStruct((B, d), jnp.float32),
    )(nr, batch_2d, w_ih, w_hh, bi_2d, bh_2d, ro_w1, rb1_2d, ro_w2, rb2_2d)


def kernel(x, edge_index, edge_attr, batch, ew1, eb1, ew2, eb2, root_w, conv_b,
           gru_w_ih, gru_w_hh, gru_b_ih, gru_b_hh,
           lstm_w_ih, lstm_w_hh, lstm_b_ih, lstm_b_hh,
           ro_w1, ro_b1, ro_w2, ro_b2):
    n, f_in = x.shape
    e = edge_index.shape[1]
    f_edge = edge_attr.shape[1]
    h_edge = ew1.shape[1]
    d = root_w.shape[0]

    epw = ((e + NW * 128 - 1) // (NW * 128)) * 128
    e_pad = epw * NW
    n_pad = ((n + 1 + NS * 8 - 1) // (NS * 8)) * (NS * 8)

    src = jnp.pad(edge_index[0], (0, e_pad - e)).reshape(e_pad // 128, 128)
    dst = jnp.pad(edge_index[1], (0, e_pad - e),
                  constant_values=n).reshape(e_pad // 128, 128)
    attr_p = jnp.pad(edge_attr, ((0, e_pad - e), (0, 0)))
    zeros_nd = jnp.zeros((n_pad, d), dtype=jnp.float32)

    h = jnp.pad(x, ((0, 0), (0, d - f_in)))
    w2r = ew2.reshape(h_edge * d, d)
    eb2_2d = eb2.reshape(d, d)
    eb1_2d = eb1.reshape(1, h_edge)

    for _ in range(LAYERS):
        hs = _sc_gather(h, src, e_pad=e_pad, d=d)
        msg = _tc_messages(attr_p, hs, ew1, eb1_2d, w2r, eb2_2d,
                           e_pad=e_pad, d=d, f_edge=f_edge, h_edge=h_edge)
        parts = _sc_scatter_add(msg, dst, zeros_nd,
                                e_pad=e_pad, n_pad=n_pad, d=d)
        h = _tc_update(parts[0, :n], parts[1, :n], h, root_w,
                       conv_b.reshape(1, d), gru_w_ih, gru_w_hh,
                       gru_b_ih.reshape(1, 3 * d), gru_b_hh.reshape(1, 3 * d),
                       n=n, d=d)

    graph_repr = _tc_set2set(h, batch.reshape(n, 1), lstm_w_ih, lstm_w_hh,
                             lstm_b_ih.reshape(1, 4 * d),
                             lstm_b_hh.reshape(1, 4 * d),
                             ro_w1, ro_b1.reshape(1, d), ro_w2,
                             ro_b2.reshape(1, d), n=n, d=d)
    return h, graph_repr


# speedup vs baseline: 3.2894x; 3.2894x over previous
import functools

import jax
import jax.numpy as jnp
from jax import lax
from jax.experimental import pallas as pl
from jax.experimental.pallas import tpu as pltpu
from jax.experimental.pallas import tpu_sc as plsc

B = 64
LAYERS = 3
STEPS = 3

NC = 2
NS = 16
NW = NC * NS


def _sc_mesh():
    return plsc.VectorSubcoreMesh(core_axis_name="c", subcore_axis_name="s",
                                  num_cores=NC, num_subcores=NS)


@functools.partial(jax.jit, static_argnames=("e_pad", "d"))
def _sc_gather(h, src2d, *, e_pad, d):
    epw = e_pad // NW
    ch = epw // 128

    @functools.partial(
        pl.kernel,
        mesh=_sc_mesh(),
        out_type=jax.ShapeDtypeStruct((e_pad, d), jnp.float32),
        scratch_types=[
            pltpu.VMEM((ch, 128), jnp.int32),
            pltpu.VMEM((128, d), jnp.float32),
            pltpu.SemaphoreType.DMA,
        ],
        compiler_params=pltpu.CompilerParams(use_tc_tiling_on_sc=False),
        name="mpnn_sc_gather",
    )
    def k(h_hbm, src_hbm, out_hbm, idx_v, rows_v, sem):
        wid = lax.axis_index("c") * NS + lax.axis_index("s")
        pltpu.sync_copy(src_hbm.at[pl.ds(wid * ch, ch)], idx_v)

        def body(j, carry):
            pltpu.async_copy(h_hbm.at[idx_v.at[j]], rows_v, sem).wait()
            pltpu.sync_copy(rows_v, out_hbm.at[pl.ds(wid * epw + j * 128, 128)])
            return carry

        lax.fori_loop(0, ch, body, 0)

    return k(h, src2d)


@functools.partial(jax.jit, static_argnames=("e_pad", "n_pad", "d"))
def _sc_scatter_add(msg, dst2d, zeros_nd, *, e_pad, n_pad, d):
    epw = e_pad // NW
    ch = epw // 128
    npt = n_pad // NS

    @functools.partial(
        pl.kernel,
        mesh=_sc_mesh(),
        out_type=jax.ShapeDtypeStruct((NC, n_pad, d), jnp.float32),
        scratch_types=[
            pltpu.VMEM((ch, 128), jnp.int32),
            pltpu.VMEM((128, d), jnp.float32),
            pltpu.VMEM_SHARED((n_pad, d), jnp.float32),
            pltpu.SemaphoreType.DMA,
        ],
        compiler_params=pltpu.CompilerParams(use_tc_tiling_on_sc=False),
        name="mpnn_sc_scatter",
    )
    def k(msg_hbm, dst_hbm, zero_hbm, out_hbm, idx_v, msg_v, acc_sh, sem):
        cid = lax.axis_index("c")
        sid = lax.axis_index("s")
        wid = cid * NS + sid
        pltpu.sync_copy(zero_hbm.at[pl.ds(sid * npt, npt)],
                        acc_sh.at[pl.ds(sid * npt, npt)])
        plsc.subcore_barrier()
        pltpu.sync_copy(dst_hbm.at[pl.ds(wid * ch, ch)], idx_v)

        def body(j, carry):
            pltpu.async_copy(msg_hbm.at[pl.ds(wid * epw + j * 128, 128)],
                             msg_v, sem).wait()
            pltpu.sync_copy(msg_v, acc_sh.at[idx_v.at[j]], add=True)
            return carry

        lax.fori_loop(0, ch, body, 0)
        plsc.subcore_barrier()
        pltpu.sync_copy(acc_sh.at[pl.ds(sid * npt, npt)],
                        out_hbm.at[cid, pl.ds(sid * npt, npt)])

    return k(msg, dst2d, zeros_nd)


def _msg_body(attr_ref, hs_ref, ew1_ref, eb1_ref, qexp_ref, w2p_ref, eb2_ref,
              out_ref):
    eh = jax.nn.relu(
        jnp.dot(attr_ref[...], ew1_ref[...],
                preferred_element_type=jnp.float32) + eb1_ref[...])
    hs = hs_ref[...]
    hs_rep = jnp.dot(hs.astype(jnp.bfloat16), qexp_ref[...],
                     preferred_element_type=jnp.float32).astype(jnp.bfloat16)
    eh_tile = jnp.tile(eh.astype(jnp.bfloat16), (1, hs.shape[1]))
    u = hs_rep * eh_tile
    out_ref[...] = (
        jnp.dot(u, w2p_ref[...], preferred_element_type=jnp.float32)
        + jnp.dot(hs, eb2_ref[...], preferred_element_type=jnp.float32))


def _tc_messages(attr_p, hs, ew1, eb1_2d, qexp, w2p, eb2_2d, *, e_pad, d,
                 f_edge, h_edge):
    et = 1024
    grid = e_pad // et
    return pl.pallas_call(
        _msg_body,
        grid=(grid,),
        in_specs=[
            pl.BlockSpec((et, f_edge), lambda i: (i, 0)),
            pl.BlockSpec((et, d), lambda i: (i, 0)),
            pl.BlockSpec((f_edge, h_edge), lambda i: (0, 0)),
            pl.BlockSpec((1, h_edge), lambda i: (0, 0)),
            pl.BlockSpec((d, h_edge * d), lambda i: (0, 0)),
            pl.BlockSpec((h_edge * d, d), lambda i: (0, 0)),
            pl.BlockSpec((d, d), lambda i: (0, 0)),
        ],
        out_specs=pl.BlockSpec((et, d), lambda i: (i, 0)),
        out_shape=jax.ShapeDtypeStruct((e_pad, d), jnp.float32),
    )(attr_p, hs, ew1, eb1_2d, qexp, w2p, eb2_2d)


def _update_body(p0_ref, p1_ref, h_ref, rw_ref, cb_ref, wih_ref, whh_ref,
                 bih_ref, bhh_ref, out_ref):
    h = h_ref[...]
    d = h.shape[1]
    agg = (p0_ref[...] + p1_ref[...]
           + jnp.dot(h, rw_ref[...], preferred_element_type=jnp.float32)
           + cb_ref[...])
    gi = jnp.dot(agg, wih_ref[...], preferred_element_type=jnp.float32) + bih_ref[...]
    gh = jnp.dot(h, whh_ref[...], preferred_element_type=jnp.float32) + bhh_ref[...]
    ir, iz, inn = gi[:, :d], gi[:, d:2 * d], gi[:, 2 * d:]
    hr, hz, hn = gh[:, :d], gh[:, d:2 * d], gh[:, 2 * d:]
    r = jax.nn.sigmoid(ir + hr)
    z = jax.nn.sigmoid(iz + hz)
    ng = jnp.tanh(inn + r * hn)
    out_ref[...] = (1.0 - z) * ng + z * h


def _tc_update(p0, p1, h, root_w, conv_b_2d, w_ih, w_hh, b_ih_2d, b_hh_2d, *, n, d):
    full = lambda *s: pl.BlockSpec(s, lambda: tuple(0 for _ in s))
    return pl.pallas_call(
        _update_body,
        in_specs=[full(n, d), full(n, d), full(n, d), full(d, d), full(1, d),
                  full(d, 3 * d), full(d, 3 * d), full(1, 3 * d), full(1, 3 * d)],
        out_specs=full(n, d),
        out_shape=jax.ShapeDtypeStruct((n, d), jnp.float32),
    )(p0, p1, h, root_w, conv_b_2d, w_ih, w_hh, b_ih_2d, b_hh_2d)


def _set2set_body(nr_ref, batch_ref, wih_ref, whh_ref, bi_ref, bh_ref,
                  row1_ref, rb1_ref, row2_ref, rb2_ref, out_ref):
    nr = nr_ref[...]
    d = nr.shape[1]
    bvec = batch_ref[...]
    gid = lax.broadcasted_iota(jnp.int32, (1, B), 1)
    m = (bvec == gid).astype(jnp.float32)
    q_star = jnp.zeros((B, 2 * d), dtype=jnp.float32)
    hl = jnp.zeros((B, d), dtype=jnp.float32)
    cl = jnp.zeros((B, d), dtype=jnp.float32)
    for _ in range(STEPS):
        g = (jnp.dot(q_star, wih_ref[...], preferred_element_type=jnp.float32)
             + jnp.dot(hl, whh_ref[...], preferred_element_type=jnp.float32)
             + bi_ref[...] + bh_ref[...])
        ig, fg, gg, og = (g[:, :d], g[:, d:2 * d], g[:, 2 * d:3 * d], g[:, 3 * d:])
        cl = jax.nn.sigmoid(fg) * cl + jax.nn.sigmoid(ig) * jnp.tanh(gg)
        hl = jax.nn.sigmoid(og) * jnp.tanh(cl)
        q = hl
        qn = jnp.dot(m, q, preferred_element_type=jnp.float32)
        e = jnp.sum(nr * qn, axis=-1, keepdims=True)
        emax = jnp.max(jnp.where(m > 0.0, e, -1e30), axis=0, keepdims=True)
        emax = jnp.where(emax < -1e29, 0.0, emax)
        ee = jnp.exp(e - jnp.sum(m * emax, axis=1, keepdims=True))
        denom = jnp.sum(m * ee, axis=0, keepdims=True)
        a = ee / (jnp.sum(m * denom, axis=1, keepdims=True) + 1e-16)
        rvec = lax.dot_general(m * a, nr, (((0,), (0,)), ((), ())),
                               preferred_element_type=jnp.float32)
        q_star = jnp.concatenate([q, rvec], axis=-1)
    hid = jax.nn.relu(
        jnp.dot(q_star, row1_ref[...], preferred_element_type=jnp.float32)
        + rb1_ref[...])
    out_ref[...] = (jnp.dot(hid, row2_ref[...], preferred_element_type=jnp.float32)
                    + rb2_ref[...])


def _tc_set2set(nr, batch_2d, w_ih, w_hh, bi_2d, bh_2d, ro_w1, rb1_2d, ro_w2,
                rb2_2d, *, n, d):
    full = lambda *s: pl.BlockSpec(s, lambda: tuple(0 for _ in s))
    return pl.pallas_call(
        _set2set_body,
        in_specs=[full(n, d), full(n, 1), full(2 * d, 4 * d), full(d, 4 * d),
                  full(1, 4 * d), full(1, 4 * d), full(2 * d, d), full(1, d),
                  full(d, d), full(1, d)],
        out_specs=full(B, d),
        out_shape=jax.ShapeDtypeStruct((B, d), jnp.float32),
    )(nr, batch_2d, w_ih, w_hh, bi_2d, bh_2d, ro_w1, rb1_2d, ro_w2, rb2_2d)


def kernel(x, edge_index, edge_attr, batch, ew1, eb1, ew2, eb2, root_w, conv_b,
           gru_w_ih, gru_w_hh, gru_b_ih, gru_b_hh,
           lstm_w_ih, lstm_w_hh, lstm_b_ih, lstm_b_hh,
           ro_w1, ro_b1, ro_w2, ro_b2):
    n, f_in = x.shape
    e = edge_index.shape[1]
    f_edge = edge_attr.shape[1]
    h_edge = ew1.shape[1]
    d = root_w.shape[0]

    epw = ((e + NW * 128 - 1) // (NW * 128)) * 128
    e_pad = epw * NW
    n_pad = ((n + 1 + NS * 8 - 1) // (NS * 8)) * (NS * 8)

    src = jnp.pad(edge_index[0], (0, e_pad - e)).reshape(e_pad // 128, 128)
    dst = jnp.pad(edge_index[1], (0, e_pad - e),
                  constant_values=n).reshape(e_pad // 128, 128)
    attr_p = jnp.pad(edge_attr, ((0, e_pad - e), (0, 0)))
    zeros_nd = jnp.zeros((n_pad, d), dtype=jnp.float32)

    h = jnp.pad(x, ((0, 0), (0, d - f_in)))
    w2p = (ew2.reshape(h_edge, d, d).transpose(1, 0, 2)
           .reshape(h_edge * d, d).astype(jnp.bfloat16))
    qexp = jnp.repeat(jnp.eye(d, dtype=jnp.bfloat16), h_edge, axis=1)
    eb2_2d = eb2.reshape(d, d)
    eb1_2d = eb1.reshape(1, h_edge)

    for _ in range(LAYERS):
        hs = _sc_gather(h, src, e_pad=e_pad, d=d)
        msg = _tc_messages(attr_p, hs, ew1, eb1_2d, qexp, w2p, eb2_2d,
                           e_pad=e_pad, d=d, f_edge=f_edge, h_edge=h_edge)
        parts = _sc_scatter_add(msg, dst, zeros_nd,
                                e_pad=e_pad, n_pad=n_pad, d=d)
        h = _tc_update(parts[0, :n], parts[1, :n], h, root_w,
                       conv_b.reshape(1, d), gru_w_ih, gru_w_hh,
                       gru_b_ih.reshape(1, 3 * d), gru_b_hh.reshape(1, 3 * d),
                       n=n, d=d)

    graph_repr = _tc_set2set(h, batch.reshape(n, 1), lstm_w_ih, lstm_w_hh,
                             lstm_b_ih.reshape(1, 4 * d),
                             lstm_b_hh.reshape(1, 4 * d),
                             ro_w1, ro_b1.reshape(1, d), ro_w2,
                             ro_b2.reshape(1, d), n=n, d=d)
    return h, graph_repr
